# baseline (speedup 1.0000x reference)
import functools

import jax
import jax.numpy as jnp
from jax import lax
from jax.experimental import pallas as pl
from jax.experimental.pallas import tpu as pltpu
from jax.experimental.pallas import tpu_sc as plsc

_D = 128
_H = 4
_N_AG, _N_MAP, _N_RT, _N_POLY = 256, 2048, 256, 1024
_K_AG, _K_MAP, _K_POLY = 128, 1024, 512
_OUT = _H * _K_AG + _K_MAP + _N_RT + _K_POLY
_OFF_MAP = _H * _K_AG
_OFF_RT = _OFF_MAP + _K_MAP
_OFF_POLY = _OFF_RT + _N_RT


def _rank_body(apr, apc, mpr, mpc, ppr, ppc, ra, rm, rp):
    def seg(rowref, colref, outref, n):
        pxr = rowref[0, 0:1, :]
        pyr = rowref[0, 1:2, :]
        drow = jnp.sqrt(pxr * pxr + pyr * pyr)
        pxc = colref[0, :, 0:1]
        pyc = colref[0, :, 1:2]
        dcol = jnp.sqrt(pxc * pxc + pyc * pyc)
        jrow = lax.broadcasted_iota(jnp.int32, (1, n), 1)
        ch = 256
        for c in range(n // ch):
            dc = dcol[c * ch:(c + 1) * ch, :]
            ic = lax.broadcasted_iota(jnp.int32, (ch, 1), 0) + c * ch
            keep = (drow < dc) | ((drow == dc) & (jrow < ic))
            cnt = jnp.sum(jnp.where(keep, 1.0, 0.0), axis=1, keepdims=True)
            outref[0, c * ch:(c + 1) * ch, :] = cnt.astype(jnp.int32)

    seg(apr, apc, ra, _N_AG)
    seg(mpr, mpc, rm, _N_MAP)
    seg(ppr, ppc, rp, _N_POLY)


def _ranks(ap, mp, pp):
    B = ap.shape[0]
    apt = jnp.swapaxes(ap, 1, 2)
    mpt = jnp.swapaxes(mp, 1, 2)
    ppt = jnp.swapaxes(pp, 1, 2)

    def spec_r(n):
        return pl.BlockSpec((1, 2, n), lambda b: (b, 0, 0))

    def spec_c(n):
        return pl.BlockSpec((1, n, 2), lambda b: (b, 0, 0))

    def spec_o(n):
        return pl.BlockSpec((1, n, 1), lambda b: (b, 0, 0))

    ra, rm, rp = pl.pallas_call(
        _rank_body,
        grid=(B,),
        in_specs=[
            spec_r(_N_AG), spec_c(_N_AG),
            spec_r(_N_MAP), spec_c(_N_MAP),
            spec_r(_N_POLY), spec_c(_N_POLY),
        ],
        out_specs=[spec_o(_N_AG), spec_o(_N_MAP), spec_o(_N_POLY)],
        out_shape=(
            jax.ShapeDtypeStruct((B, _N_AG, 1), jnp.int32),
            jax.ShapeDtypeStruct((B, _N_MAP, 1), jnp.int32),
            jax.ShapeDtypeStruct((B, _N_POLY, 1), jnp.int32),
        ),
        compiler_params=pltpu.CompilerParams(
            dimension_semantics=("arbitrary",)),
    )(apt, ap, mpt, mp, ppt, pp)
    return (ra.reshape(B * _N_AG), rm.reshape(B * _N_MAP),
            rp.reshape(B * _N_POLY))


def _sc_gather(ra, rm, rp, af, mf, rf, pf, B):
    mesh = plsc.VectorSubcoreMesh(core_axis_name="c", subcore_axis_name="s")
    nb = B // 32

    @functools.partial(
        pl.kernel,
        out_type=jax.ShapeDtypeStruct((B * _OUT, _D), jnp.float32),
        mesh=mesh,
        compiler_params=pltpu.CompilerParams(needs_layout_passes=False),
        scratch_types=[
            pltpu.VMEM((_N_AG,), jnp.int32),
            pltpu.VMEM((_N_MAP,), jnp.int32),
            pltpu.VMEM((_N_POLY,), jnp.int32),
            pltpu.VMEM((_H * _K_AG,), jnp.int32),
            pltpu.VMEM((_K_MAP,), jnp.int32),
            pltpu.VMEM((_K_POLY,), jnp.int32),
            pltpu.VMEM((2, 128, _D), jnp.float32),
            pltpu.SemaphoreType.DMA,
            pltpu.SemaphoreType.DMA,
        ],
    )
    def k(ra_h, rm_h, rp_h, af_h, mf_h, rf_h, pf_h, out_h,
          ra_v, rm_v, rp_v, ia_v, im_v, ip_v, buf, sem_g, sem_o):
        wid = lax.axis_index("s") * 2 + lax.axis_index("c")
        lane = lax.iota(jnp.int32, 16)
        for bi in range(nb):
            b = wid * nb + bi
            pltpu.sync_copy(ra_h.at[pl.ds(b * _N_AG, _N_AG)], ra_v)
            pltpu.sync_copy(rm_h.at[pl.ds(b * _N_MAP, _N_MAP)], rm_v)
            pltpu.sync_copy(rp_h.at[pl.ds(b * _N_POLY, _N_POLY)], rp_v)

            ab = b * (_H * _N_AG)

            def ag_body(i, _):
                r = ra_v[pl.ds(i * 16, 16)]
                m = r < _K_AG
                src = lane + i * 16 + ab
                for h in range(_H):
                    plsc.store_scatter(
                        ia_v, [r + h * _K_AG], src + h * _N_AG, mask=m)
                return 0

            lax.fori_loop(0, _N_AG // 16, ag_body, 0)

            mb = b * _N_MAP

            def mp_body(i, _):
                r = rm_v[pl.ds(i * 16, 16)]
                m = r < _K_MAP
                plsc.store_scatter(im_v, [r], lane + i * 16 + mb, mask=m)
                return 0

            lax.fori_loop(0, _N_MAP // 16, mp_body, 0)

            pb = b * _N_POLY

            def pp_body(i, _):
                r = rp_v[pl.ds(i * 16, 16)]
                m = r < _K_POLY
                plsc.store_scatter(ip_v, [r], lane + i * 16 + pb, mask=m)
                return 0

            lax.fori_loop(0, _N_POLY // 16, pp_body, 0)

            ob = b * _OUT

            def run_seg(idx_ref, nch, table, base):
                def body(j, _):
                    pltpu.async_copy(
                        table.at[idx_ref.at[pl.ds(j * 128, 128)]],
                        buf.at[0], sem_g).wait()
                    pltpu.sync_copy(
                        buf.at[0], out_h.at[pl.ds(base + j * 128, 128)])
                    return 0

                lax.fori_loop(0, nch, body, 0)

            run_seg(ia_v, _H * _K_AG // 128, af_h, ob)
            run_seg(im_v, _K_MAP // 128, mf_h, ob + _OFF_MAP)
            run_seg(ip_v, _K_POLY // 128, pf_h, ob + _OFF_POLY)

            def rt_body(j, _):
                pltpu.sync_copy(
                    rf_h.at[pl.ds(b * _N_RT + j * 128, 128)], buf.at[1])
                pltpu.sync_copy(
                    buf.at[1], out_h.at[pl.ds(ob + _OFF_RT + j * 128, 128)])
                return 0

            lax.fori_loop(0, _N_RT // 128, rt_body, 0)

    return k(ra, rm, rp, af, mf, rf, pf)


def kernel(agent_feats, agent_poses, map_feats, map_poses, route_feats,
           polygon_feats, polygon_poses):
    B = agent_feats.shape[0]
    ra, rm, rp = _ranks(agent_poses, map_poses, polygon_poses)
    out = _sc_gather(
        ra, rm, rp,
        agent_feats.reshape(B * _H * _N_AG, _D),
        map_feats.reshape(B * _N_MAP, _D),
        route_feats.reshape(B * _N_RT, _D),
        polygon_feats.reshape(B * _N_POLY, _D),
        B)
    return out.reshape(B, _OUT, _D)


# speedup vs baseline: 2.9167x; 2.9167x over previous
import functools

import jax
import jax.numpy as jnp
from jax import lax
from jax.experimental import pallas as pl
from jax.experimental.pallas import tpu as pltpu
from jax.experimental.pallas import tpu_sc as plsc

_D = 128
_H = 4
_N_AG, _N_MAP, _N_RT, _N_POLY = 256, 2048, 256, 1024
_K_AG, _K_MAP, _K_POLY = 128, 1024, 512
_OUT = _H * _K_AG + _K_MAP + _N_RT + _K_POLY
_OFF_MAP = _H * _K_AG
_OFF_RT = _OFF_MAP + _K_MAP
_OFF_POLY = _OFF_RT + _N_RT
_NK = _N_MAP + _N_POLY + _N_AG
_KO_MAP, _KO_POLY, _KO_AG = 0, _N_MAP, _N_MAP + _N_POLY


def _keys_body(mp_ref, pp_ref, ap_ref, out_ref):
    def seg(ref, lo, n):
        px = ref[:, 0, :]
        py = ref[:, 1, :]
        d = jnp.sqrt(px * px + py * py)
        out_ref[:, lo:lo + n] = lax.bitcast_convert_type(d, jnp.int32)

    seg(mp_ref, _KO_MAP, _N_MAP)
    seg(pp_ref, _KO_POLY, _N_POLY)
    seg(ap_ref, _KO_AG, _N_AG)


def _keys(ap, mp, pp):
    B = ap.shape[0]
    apt = jnp.swapaxes(ap, 1, 2)
    mpt = jnp.swapaxes(mp, 1, 2)
    ppt = jnp.swapaxes(pp, 1, 2)
    bb = 16

    def spec(n):
        return pl.BlockSpec((bb, 2, n), lambda b: (b, 0, 0))

    keys = pl.pallas_call(
        _keys_body,
        grid=(B // bb,),
        in_specs=[spec(_N_MAP), spec(_N_POLY), spec(_N_AG)],
        out_specs=pl.BlockSpec((bb, _NK), lambda b: (b, 0)),
        out_shape=jax.ShapeDtypeStruct((B, _NK), jnp.int32),
        compiler_params=pltpu.CompilerParams(
            dimension_semantics=("arbitrary",)),
    )(mpt, ppt, apt)
    return keys.reshape(B * _NK)


def _sc_gather(keys, af, mf, rf, pf, B):
    mesh = plsc.VectorSubcoreMesh(core_axis_name="c", subcore_axis_name="s")
    nb = B // 32

    @functools.partial(
        pl.kernel,
        out_type=jax.ShapeDtypeStruct((B * _OUT, _D), jnp.float32),
        mesh=mesh,
        compiler_params=pltpu.CompilerParams(needs_layout_passes=False),
        scratch_types=[
            pltpu.VMEM((_NK,), jnp.int32),
            pltpu.VMEM((_N_MAP,), jnp.int32),
            pltpu.VMEM((_N_MAP,), jnp.int32),
            pltpu.VMEM((_N_MAP,), jnp.int32),
            pltpu.VMEM((_N_MAP,), jnp.int32),
            pltpu.VMEM((256,), jnp.int32),
            pltpu.VMEM((_K_MAP,), jnp.int32),
            pltpu.VMEM((_K_POLY,), jnp.int32),
            pltpu.VMEM((_H * _K_AG,), jnp.int32),
            pltpu.VMEM((2, 128, _D), jnp.float32),
            pltpu.SemaphoreType.DMA,
            pltpu.SemaphoreType.DMA,
        ],
    )
    def k(keys_h, af_h, mf_h, rf_h, pf_h, out_h,
          keys_v, k0, i0, k1, i1, off, idx_m, idx_p, idx_a, buf,
          sem_g, sem_o):
        wid = lax.axis_index("s") * 2 + lax.axis_index("c")
        lane = lax.iota(jnp.int32, 16)

        def radix_argsort(ko, n):
            for p in range(4):
                if p == 0:
                    kin, iin = None, None
                elif p % 2 == 1:
                    kin, iin = k0, i0
                else:
                    kin, iin = k1, i1
                kout, iout = (k0, i0) if p % 2 == 0 else (k1, i1)
                sh = 8 * p

                def hz(i, _):
                    off[pl.ds(i * 16, 16)] = jnp.zeros((16,), jnp.int32)
                    return 0

                lax.fori_loop(0, 16, hz, 0)

                def hb(i, _):
                    if p == 0:
                        kk = keys_v[pl.ds(ko + i * 16, 16)]
                    else:
                        kk = kin[pl.ds(i * 16, 16)]
                    d = lax.shift_right_logical(kk, sh) & 255
                    occ, lastm = plsc.scan_count(d)
                    plsc.addupdate_scatter(off, [d], occ, mask=lastm)
                    return 0

                lax.fori_loop(0, n // 16, hb, 0)

                def pb(i, carry):
                    h = off[pl.ds(i * 16, 16)]
                    c = plsc.cumsum(h)
                    off[pl.ds(i * 16, 16)] = c - h + carry
                    return carry + jnp.sum(h)

                lax.fori_loop(0, 16, pb, jnp.int32(0))

                def mb(i, _):
                    if p == 0:
                        kk = keys_v[pl.ds(ko + i * 16, 16)]
                        v = lane + i * 16
                    else:
                        kk = kin[pl.ds(i * 16, 16)]
                        v = iin[pl.ds(i * 16, 16)]
                    d = lax.shift_right_logical(kk, sh) & 255
                    occ, lastm = plsc.scan_count(d)
                    base = plsc.load_gather(off, [d])
                    pos = base + occ - 1
                    if p != 3:
                        plsc.store_scatter(kout, [pos], kk)
                    plsc.store_scatter(iout, [pos], v)
                    plsc.addupdate_scatter(off, [d], occ, mask=lastm)
                    return 0

                lax.fori_loop(0, n // 16, mb, 0)

        def batch_body(bi, _):
            b = wid * nb + bi
            pltpu.sync_copy(keys_h.at[pl.ds(b * _NK, _NK)], keys_v)

            radix_argsort(_KO_MAP, _N_MAP)
            mb_base = b * _N_MAP

            def mi(i, _):
                idx_m[pl.ds(i * 16, 16)] = i1[pl.ds(i * 16, 16)] + mb_base
                return 0

            lax.fori_loop(0, _K_MAP // 16, mi, 0)

            radix_argsort(_KO_POLY, _N_POLY)
            pb_base = b * _N_POLY

            def pi(i, _):
                idx_p[pl.ds(i * 16, 16)] = i1[pl.ds(i * 16, 16)] + pb_base
                return 0

            lax.fori_loop(0, _K_POLY // 16, pi, 0)

            radix_argsort(_KO_AG, _N_AG)
            ab_base = b * (_H * _N_AG)
            for h in range(_H):
                def ai(i, _):
                    idx_a[pl.ds(h * _K_AG + i * 16, 16)] = (
                        i1[pl.ds(i * 16, 16)] + (ab_base + h * _N_AG))
                    return 0

                lax.fori_loop(0, _K_AG // 16, ai, 0)

            ob = b * _OUT

            def run_seg(idx_ref, nch, table, base):
                def body(j, _):
                    pltpu.async_copy(
                        table.at[idx_ref.at[pl.ds(j * 128, 128)]],
                        buf.at[0], sem_g).wait()
                    pltpu.sync_copy(
                        buf.at[0], out_h.at[pl.ds(base + j * 128, 128)])
                    return 0

                lax.fori_loop(0, nch, body, 0)

            run_seg(idx_a, _H * _K_AG // 128, af_h, ob)
            run_seg(idx_m, _K_MAP // 128, mf_h, ob + _OFF_MAP)
            run_seg(idx_p, _K_POLY // 128, pf_h, ob + _OFF_POLY)

            def rt_body(j, _):
                pltpu.sync_copy(
                    rf_h.at[pl.ds(b * _N_RT + j * 128, 128)], buf.at[1])
                pltpu.sync_copy(
                    buf.at[1], out_h.at[pl.ds(ob + _OFF_RT + j * 128, 128)])
                return 0

            lax.fori_loop(0, _N_RT // 128, rt_body, 0)
            return 0

        lax.fori_loop(0, nb, batch_body, 0)

    return k(keys, af, mf, rf, pf)


def kernel(agent_feats, agent_poses, map_feats, map_poses, route_feats,
           polygon_feats, polygon_poses):
    B = agent_feats.shape[0]
    keys = _keys(agent_poses, map_poses, polygon_poses)
    out = _sc_gather(
        keys,
        agent_feats.reshape(B * _H * _N_AG, _D),
        map_feats.reshape(B * _N_MAP, _D),
        route_feats.reshape(B * _N_RT, _D),
        polygon_feats.reshape(B * _N_POLY, _D),
        B)
    return out.reshape(B, _OUT, _D)


# speedup vs baseline: 3.7518x; 1.2863x over previous
import functools

import jax
import jax.numpy as jnp
from jax import lax
from jax.experimental import pallas as pl
from jax.experimental.pallas import tpu as pltpu
from jax.experimental.pallas import tpu_sc as plsc

_D = 128
_H = 4
_N_AG, _N_MAP, _N_RT, _N_POLY = 256, 2048, 256, 1024
_K_AG, _K_MAP, _K_POLY = 128, 1024, 512
_OUT = _H * _K_AG + _K_MAP + _N_RT + _K_POLY
_OFF_MAP = _H * _K_AG
_OFF_RT = _OFF_MAP + _K_MAP
_OFF_POLY = _OFF_RT + _N_RT
_NK = _N_MAP + _N_POLY + _N_AG
_KO_MAP, _KO_POLY, _KO_AG = 0, _N_MAP, _N_MAP + _N_POLY


def _keys_body(mp_ref, pp_ref, ap_ref, out_ref):
    def seg(ref, lo, n):
        px = ref[:, 0, :]
        py = ref[:, 1, :]
        d = jnp.sqrt(px * px + py * py)
        out_ref[:, lo:lo + n] = lax.bitcast_convert_type(d, jnp.int32)

    seg(mp_ref, _KO_MAP, _N_MAP)
    seg(pp_ref, _KO_POLY, _N_POLY)
    seg(ap_ref, _KO_AG, _N_AG)


def _keys(ap, mp, pp):
    B = ap.shape[0]
    apt = jnp.swapaxes(ap, 1, 2)
    mpt = jnp.swapaxes(mp, 1, 2)
    ppt = jnp.swapaxes(pp, 1, 2)
    bb = 16

    def spec(n):
        return pl.BlockSpec((bb, 2, n), lambda b: (b, 0, 0))

    keys = pl.pallas_call(
        _keys_body,
        grid=(B // bb,),
        in_specs=[spec(_N_MAP), spec(_N_POLY), spec(_N_AG)],
        out_specs=pl.BlockSpec((bb, _NK), lambda b: (b, 0)),
        out_shape=jax.ShapeDtypeStruct((B, _NK), jnp.int32),
        compiler_params=pltpu.CompilerParams(
            dimension_semantics=("arbitrary",)),
    )(mpt, ppt, apt)
    return keys.reshape(B * _NK)


def _sc_gather(keys, af, mf, rf, pf, B):
    mesh = plsc.VectorSubcoreMesh(core_axis_name="c", subcore_axis_name="s")
    nb = B // 32

    @functools.partial(
        pl.kernel,
        out_type=jax.ShapeDtypeStruct((B * _OUT, _D), jnp.float32),
        mesh=mesh,
        compiler_params=pltpu.CompilerParams(needs_layout_passes=False),
        scratch_types=[
            pltpu.VMEM((_NK,), jnp.int32),
            pltpu.VMEM((_N_MAP,), jnp.int32),
            pltpu.VMEM((_N_MAP,), jnp.int32),
            pltpu.VMEM((_N_MAP,), jnp.int32),
            pltpu.VMEM((_N_MAP,), jnp.int32),
            pltpu.VMEM((256,), jnp.int32),
            pltpu.VMEM((_K_MAP,), jnp.int32),
            pltpu.VMEM((_K_POLY,), jnp.int32),
            pltpu.VMEM((_H * _K_AG,), jnp.int32),
            pltpu.VMEM((6, 128, _D), jnp.float32),
            pltpu.SemaphoreType.DMA,
            pltpu.SemaphoreType.DMA,
        ],
    )
    def k(keys_h, af_h, mf_h, rf_h, pf_h, out_h,
          keys_v, k0, i0, k1, i1, off, idx_m, idx_p, idx_a, buf,
          sem_g, sem_o):
        wid = lax.axis_index("s") * 2 + lax.axis_index("c")
        lane = lax.iota(jnp.int32, 16)

        def radix_argsort(ko, n):
            for p in range(4):
                if p == 0:
                    kin, iin = None, None
                elif p % 2 == 1:
                    kin, iin = k0, i0
                else:
                    kin, iin = k1, i1
                kout, iout = (k0, i0) if p % 2 == 0 else (k1, i1)
                sh = 8 * p

                def hz(i, _):
                    off[pl.ds(i * 16, 16)] = jnp.zeros((16,), jnp.int32)
                    return 0

                lax.fori_loop(0, 16, hz, 0)

                def hb(i, _):
                    if p == 0:
                        kk = keys_v[pl.ds(ko + i * 16, 16)]
                    else:
                        kk = kin[pl.ds(i * 16, 16)]
                    d = lax.shift_right_logical(kk, sh) & 255
                    occ, lastm = plsc.scan_count(d)
                    plsc.addupdate_scatter(off, [d], occ, mask=lastm)
                    return 0

                lax.fori_loop(0, n // 16, hb, 0)

                def pb(i, carry):
                    h = off[pl.ds(i * 16, 16)]
                    c = plsc.cumsum(h)
                    off[pl.ds(i * 16, 16)] = c - h + carry
                    return carry + jnp.sum(h)

                lax.fori_loop(0, 16, pb, jnp.int32(0))

                def mb(i, _):
                    if p == 0:
                        kk = keys_v[pl.ds(ko + i * 16, 16)]
                        v = lane + i * 16
                    else:
                        kk = kin[pl.ds(i * 16, 16)]
                        v = iin[pl.ds(i * 16, 16)]
                    d = lax.shift_right_logical(kk, sh) & 255
                    occ, lastm = plsc.scan_count(d)
                    base = plsc.load_gather(off, [d])
                    pos = base + occ - 1
                    if p != 3:
                        plsc.store_scatter(kout, [pos], kk)
                    plsc.store_scatter(iout, [pos], v)
                    plsc.addupdate_scatter(off, [d], occ, mask=lastm)
                    return 0

                lax.fori_loop(0, n // 16, mb, 0)

        def batch_body(bi, _):
            b = wid * nb + bi
            pltpu.sync_copy(keys_h.at[pl.ds(b * _NK, _NK)], keys_v)
            ob = b * _OUT

            def build_idx(dst, count, base):
                def bd(i, _):
                    dst[pl.ds(i * 16, 16)] = i1[pl.ds(i * 16, 16)] + base
                    return 0

                lax.fori_loop(0, count // 16, bd, 0)

            chunks = (
                [("g", idx_m, j, mf_h, ob + _OFF_MAP + j * 128)
                 for j in range(8)]
                + [("g", idx_p, j, pf_h, ob + _OFF_POLY + j * 128)
                   for j in range(4)]
                + [("g", idx_a, j, af_h, ob + j * 128) for j in range(4)]
                + [("r", None, j, rf_h, ob + _OFF_RT + j * 128)
                   for j in range(2)]
            )
            n_ch = len(chunks)
            gh = {}
            wh = {}

            def fire(j):
                kind, idx_ref, jj, table, _ = chunks[j]
                slot = buf.at[j % 6]
                if kind == "g":
                    gh[j] = pltpu.async_copy(
                        table.at[idx_ref.at[pl.ds(jj * 128, 128)]],
                        slot, sem_g)
                else:
                    gh[j] = pltpu.async_copy(
                        table.at[pl.ds(b * _N_RT + jj * 128, 128)],
                        slot, sem_g)

            radix_argsort(_KO_MAP, _N_MAP)
            build_idx(idx_m, _K_MAP, b * _N_MAP)
            fire(0)
            fire(1)
            fire(2)
            fire(3)
            radix_argsort(_KO_POLY, _N_POLY)
            build_idx(idx_p, _K_POLY, b * _N_POLY)
            radix_argsort(_KO_AG, _N_AG)
            ab_base = b * (_H * _N_AG)
            for h in range(_H):
                def ai(i, _):
                    idx_a[pl.ds(h * _K_AG + i * 16, 16)] = (
                        i1[pl.ds(i * 16, 16)] + (ab_base + h * _N_AG))
                    return 0

                lax.fori_loop(0, _K_AG // 16, ai, 0)

            for j in range(n_ch):
                nxt = j + 4
                if nxt < n_ch:
                    if nxt >= 6:
                        wh[nxt - 6].wait()
                    fire(nxt)
                gh[j].wait()
                wh[j] = pltpu.async_copy(
                    buf.at[j % 6], out_h.at[pl.ds(chunks[j][4], 128)], sem_o)
            for j in range(n_ch - 6, n_ch):
                wh[j].wait()
            return 0

        lax.fori_loop(0, nb, batch_body, 0)

    return k(keys, af, mf, rf, pf)


def kernel(agent_feats, agent_poses, map_feats, map_poses, route_feats,
           polygon_feats, polygon_poses):
    B = agent_feats.shape[0]
    keys = _keys(agent_poses, map_poses, polygon_poses)
    out = _sc_gather(
        keys,
        agent_feats.reshape(B * _H * _N_AG, _D),
        map_feats.reshape(B * _N_MAP, _D),
        route_feats.reshape(B * _N_RT, _D),
        polygon_feats.reshape(B * _N_POLY, _D),
        B)
    return out.reshape(B, _OUT, _D)


# speedup vs baseline: 4.4080x; 1.1749x over previous
import functools

import jax
import jax.numpy as jnp
from jax import lax
from jax.experimental import pallas as pl
from jax.experimental.pallas import tpu as pltpu
from jax.experimental.pallas import tpu_sc as plsc

_D = 128
_H = 4
_N_AG, _N_MAP, _N_RT, _N_POLY = 256, 2048, 256, 1024
_K_AG, _K_MAP, _K_POLY = 128, 1024, 512
_OUT = _H * _K_AG + _K_MAP + _N_RT + _K_POLY
_OFF_MAP = _H * _K_AG
_OFF_RT = _OFF_MAP + _K_MAP
_OFF_POLY = _OFF_RT + _N_RT
_NK = _N_MAP + _N_POLY + _N_AG
_KO_MAP, _KO_POLY, _KO_AG = 0, _N_MAP, _N_MAP + _N_POLY


def _keys_body(mp_ref, pp_ref, ap_ref, out_ref):
    def seg(ref, lo, n):
        px = ref[:, 0, :]
        py = ref[:, 1, :]
        d = jnp.sqrt(px * px + py * py)
        out_ref[:, lo:lo + n] = lax.bitcast_convert_type(d, jnp.int32)

    seg(mp_ref, _KO_MAP, _N_MAP)
    seg(pp_ref, _KO_POLY, _N_POLY)
    seg(ap_ref, _KO_AG, _N_AG)


def _keys(ap, mp, pp):
    B = ap.shape[0]
    apt = jnp.swapaxes(ap, 1, 2)
    mpt = jnp.swapaxes(mp, 1, 2)
    ppt = jnp.swapaxes(pp, 1, 2)
    bb = 16

    def spec(n):
        return pl.BlockSpec((bb, 2, n), lambda b: (b, 0, 0))

    keys = pl.pallas_call(
        _keys_body,
        grid=(B // bb,),
        in_specs=[spec(_N_MAP), spec(_N_POLY), spec(_N_AG)],
        out_specs=pl.BlockSpec((bb, _NK), lambda b: (b, 0)),
        out_shape=jax.ShapeDtypeStruct((B, _NK), jnp.int32),
        compiler_params=pltpu.CompilerParams(
            dimension_semantics=("arbitrary",)),
    )(mpt, ppt, apt)
    return keys.reshape(B * _NK)


def _sc_gather(keys, af, mf, rf, pf, B):
    mesh = plsc.VectorSubcoreMesh(core_axis_name="c", subcore_axis_name="s")
    nb = B // 32

    @functools.partial(
        pl.kernel,
        out_type=jax.ShapeDtypeStruct((B * _OUT, _D), jnp.float32),
        mesh=mesh,
        compiler_params=pltpu.CompilerParams(needs_layout_passes=False),
        scratch_types=[
            pltpu.VMEM((_NK,), jnp.int32),
            pltpu.VMEM((_N_MAP,), jnp.int32),
            pltpu.VMEM((_N_MAP,), jnp.int32),
            pltpu.VMEM((_N_MAP,), jnp.int32),
            pltpu.VMEM((_N_MAP,), jnp.int32),
            pltpu.VMEM((256,), jnp.int32),
            pltpu.VMEM((_K_MAP,), jnp.int32),
            pltpu.VMEM((_K_MAP,), jnp.int32),
            pltpu.VMEM((_K_POLY,), jnp.int32),
            pltpu.VMEM((_K_POLY,), jnp.int32),
            pltpu.VMEM((_H * _K_AG,), jnp.int32),
            pltpu.VMEM((_H * _K_AG,), jnp.int32),
            pltpu.VMEM((6, 128, _D), jnp.float32),
            pltpu.SemaphoreType.DMA,
            pltpu.SemaphoreType.DMA,
        ],
    )
    def k(keys_h, af_h, mf_h, rf_h, pf_h, out_h,
          keys_v, k0, i0, k1, i1, off, idx_m0, idx_m1, idx_p0, idx_p1,
          idx_a0, idx_a1, buf, sem_g, sem_o):
        idx_m = (idx_m0, idx_m1)
        idx_p = (idx_p0, idx_p1)
        idx_a = (idx_a0, idx_a1)
        wid = lax.axis_index("s") * 2 + lax.axis_index("c")
        lane = lax.iota(jnp.int32, 16)
        nslot = 6
        n_ch = 18

        def radix_pass(ko, n, p):
            if p == 0:
                kin, iin = None, None
            elif p % 2 == 1:
                kin, iin = k0, i0
            else:
                kin, iin = k1, i1
            kout, iout = (k0, i0) if p % 2 == 0 else (k1, i1)
            sh = 8 * p

            def hz(i, _):
                off[pl.ds(i * 16, 16)] = jnp.zeros((16,), jnp.int32)
                return 0

            lax.fori_loop(0, 16, hz, 0)

            def hb(i, _):
                if p == 0:
                    kk = keys_v[pl.ds(ko + i * 16, 16)]
                else:
                    kk = kin[pl.ds(i * 16, 16)]
                d = lax.shift_right_logical(kk, sh) & 255
                occ, lastm = plsc.scan_count(d)
                plsc.addupdate_scatter(off, [d], occ, mask=lastm)
                return 0

            lax.fori_loop(0, n // 16, hb, 0)

            def pb(i, carry):
                h = off[pl.ds(i * 16, 16)]
                c = plsc.cumsum(h)
                off[pl.ds(i * 16, 16)] = c - h + carry
                return carry + jnp.sum(h)

            lax.fori_loop(0, 16, pb, jnp.int32(0))

            def mb(i, _):
                if p == 0:
                    kk = keys_v[pl.ds(ko + i * 16, 16)]
                    v = lane + i * 16
                else:
                    kk = kin[pl.ds(i * 16, 16)]
                    v = iin[pl.ds(i * 16, 16)]
                d = lax.shift_right_logical(kk, sh) & 255
                occ, lastm = plsc.scan_count(d)
                base = plsc.load_gather(off, [d])
                pos = base + occ - 1
                if p != 3:
                    plsc.store_scatter(kout, [pos], kk)
                plsc.store_scatter(iout, [pos], v)
                plsc.addupdate_scatter(off, [d], occ, mask=lastm)
                return 0

            lax.fori_loop(0, n // 16, mb, 0)

        def build_idx(dst, count, base):
            def bd(i, _):
                dst[pl.ds(i * 16, 16)] = i1[pl.ds(i * 16, 16)] + base
                return 0

            lax.fori_loop(0, count // 16, bd, 0)

        def build_idx_agent(dst, b):
            ab_base = b * (_H * _N_AG)
            for h in range(_H):
                def ai(i, _):
                    dst[pl.ds(h * _K_AG + i * 16, 16)] = (
                        i1[pl.ds(i * 16, 16)] + (ab_base + h * _N_AG))
                    return 0

                lax.fori_loop(0, _K_AG // 16, ai, 0)

        def compute_steps(bi):
            b = wid * nb + bi
            im, ip, ia = idx_m[bi % 2], idx_p[bi % 2], idx_a[bi % 2]
            return [
                lambda: (pltpu.sync_copy(
                    keys_h.at[pl.ds(b * _NK, _NK)], keys_v),
                    radix_pass(_KO_MAP, _N_MAP, 0))[-1],
                lambda: radix_pass(_KO_MAP, _N_MAP, 1),
                lambda: radix_pass(_KO_MAP, _N_MAP, 2),
                lambda: (radix_pass(_KO_MAP, _N_MAP, 3),
                         build_idx(im, _K_MAP, b * _N_MAP))[-1],
                lambda: (radix_pass(_KO_POLY, _N_POLY, 0),
                         radix_pass(_KO_POLY, _N_POLY, 1))[-1],
                lambda: (radix_pass(_KO_POLY, _N_POLY, 2),
                         radix_pass(_KO_POLY, _N_POLY, 3),
                         build_idx(ip, _K_POLY, b * _N_POLY))[-1],
                lambda: (radix_pass(_KO_AG, _N_AG, 0),
                         radix_pass(_KO_AG, _N_AG, 1))[-1],
                lambda: (radix_pass(_KO_AG, _N_AG, 2),
                         radix_pass(_KO_AG, _N_AG, 3),
                         build_idx_agent(ia, b))[-1],
            ]

        def chunks_for(bi):
            b = wid * nb + bi
            ob = b * _OUT
            im, ip, ia = idx_m[bi % 2], idx_p[bi % 2], idx_a[bi % 2]
            return (
                [("g", im, j, mf_h, ob + _OFF_MAP + j * 128, b)
                 for j in range(8)]
                + [("g", ip, j, pf_h, ob + _OFF_POLY + j * 128, b)
                   for j in range(4)]
                + [("g", ia, j, af_h, ob + j * 128, b) for j in range(4)]
                + [("r", None, j, rf_h, ob + _OFF_RT + j * 128, b)
                   for j in range(2)]
            )

        all_chunks = []
        for bi in range(nb):
            all_chunks.extend(chunks_for(bi))
        n_total = len(all_chunks)
        gh = {}
        wh = {}

        def fire(J):
            kind, idx_ref, jj, table, _, b = all_chunks[J]
            slot = buf.at[J % nslot]
            if kind == "g":
                gh[J] = pltpu.async_copy(
                    table.at[idx_ref.at[pl.ds(jj * 128, 128)]], slot, sem_g)
            else:
                gh[J] = pltpu.async_copy(
                    table.at[pl.ds(b * _N_RT + jj * 128, 128)], slot, sem_g)

        for step in compute_steps(0):
            step()
        for J in range(4):
            fire(J)

        for J in range(n_total):
            bi, j = divmod(J, n_ch)
            nxt = J + 4
            if nxt < n_total:
                if nxt >= nslot:
                    wh[nxt - nslot].wait()
                fire(nxt)
            gh[J].wait()
            wh[J] = pltpu.async_copy(
                buf.at[J % nslot],
                out_h.at[pl.ds(all_chunks[J][4], 128)], sem_o)
            if bi + 1 < nb and j % 2 == 0 and j // 2 < 8:
                compute_steps(bi + 1)[j // 2]()
        for J in range(n_total - nslot, n_total):
            wh[J].wait()

    return k(keys, af, mf, rf, pf)


def kernel(agent_feats, agent_poses, map_feats, map_poses, route_feats,
           polygon_feats, polygon_poses):
    B = agent_feats.shape[0]
    keys = _keys(agent_poses, map_poses, polygon_poses)
    out = _sc_gather(
        keys,
        agent_feats.reshape(B * _H * _N_AG, _D),
        map_feats.reshape(B * _N_MAP, _D),
        route_feats.reshape(B * _N_RT, _D),
        polygon_feats.reshape(B * _N_POLY, _D),
        B)
    return out.reshape(B, _OUT, _D)


# speedup vs baseline: 4.4119x; 1.0009x over previous
import functools

import jax
import jax.numpy as jnp
from jax import lax
from jax.experimental import pallas as pl
from jax.experimental.pallas import tpu as pltpu
from jax.experimental.pallas import tpu_sc as plsc

_D = 128
_H = 4
_N_AG, _N_MAP, _N_RT, _N_POLY = 256, 2048, 256, 1024
_K_AG, _K_MAP, _K_POLY = 128, 1024, 512
_OUT = _H * _K_AG + _K_MAP + _N_RT + _K_POLY
_OFF_MAP = _H * _K_AG
_OFF_RT = _OFF_MAP + _K_MAP
_OFF_POLY = _OFF_RT + _N_RT
_NK = _N_MAP + _N_POLY + _N_AG
_KO_MAP, _KO_POLY, _KO_AG = 0, _N_MAP, _N_MAP + _N_POLY


def _keys_body(mp_ref, pp_ref, ap_ref, out_ref):
    def seg(ref, lo, n):
        px = ref[:, 0, :]
        py = ref[:, 1, :]
        d = jnp.sqrt(px * px + py * py)
        out_ref[:, lo:lo + n] = lax.bitcast_convert_type(d, jnp.int32)

    seg(mp_ref, _KO_MAP, _N_MAP)
    seg(pp_ref, _KO_POLY, _N_POLY)
    seg(ap_ref, _KO_AG, _N_AG)


def _keys(ap, mp, pp):
    B = ap.shape[0]
    apt = jnp.swapaxes(ap, 1, 2)
    mpt = jnp.swapaxes(mp, 1, 2)
    ppt = jnp.swapaxes(pp, 1, 2)
    bb = 16

    def spec(n):
        return pl.BlockSpec((bb, 2, n), lambda b: (b, 0, 0))

    keys = pl.pallas_call(
        _keys_body,
        grid=(B // bb,),
        in_specs=[spec(_N_MAP), spec(_N_POLY), spec(_N_AG)],
        out_specs=pl.BlockSpec((bb, _NK), lambda b: (b, 0)),
        out_shape=jax.ShapeDtypeStruct((B, _NK), jnp.int32),
        compiler_params=pltpu.CompilerParams(
            dimension_semantics=("arbitrary",)),
    )(mpt, ppt, apt)
    return keys.reshape(B * _NK)


def _sc_gather(keys, af, mf, rf, pf, B):
    mesh = plsc.VectorSubcoreMesh(core_axis_name="c", subcore_axis_name="s")
    nb = B // 32

    @functools.partial(
        pl.kernel,
        out_type=jax.ShapeDtypeStruct((B * _OUT, _D), jnp.float32),
        mesh=mesh,
        compiler_params=pltpu.CompilerParams(needs_layout_passes=False),
        scratch_types=[
            pltpu.VMEM((_NK,), jnp.int32),
            pltpu.VMEM((_N_MAP,), jnp.int32),
            pltpu.VMEM((_N_MAP,), jnp.int32),
            pltpu.VMEM((_N_MAP,), jnp.int32),
            pltpu.VMEM((_N_MAP,), jnp.int32),
            pltpu.VMEM((256,), jnp.int32),
            pltpu.VMEM((_K_MAP,), jnp.int32),
            pltpu.VMEM((_K_MAP,), jnp.int32),
            pltpu.VMEM((_K_POLY,), jnp.int32),
            pltpu.VMEM((_K_POLY,), jnp.int32),
            pltpu.VMEM((_H * _K_AG,), jnp.int32),
            pltpu.VMEM((_H * _K_AG,), jnp.int32),
            pltpu.VMEM((6, 128, _D), jnp.float32),
            pltpu.SemaphoreType.DMA,
            pltpu.SemaphoreType.DMA,
        ],
    )
    def k(keys_h, af_h, mf_h, rf_h, pf_h, out_h,
          keys_v, k0, i0, k1, i1, off, idx_m0, idx_m1, idx_p0, idx_p1,
          idx_a0, idx_a1, buf, sem_g, sem_o):
        idx_m = (idx_m0, idx_m1)
        idx_p = (idx_p0, idx_p1)
        idx_a = (idx_a0, idx_a1)
        wid = lax.axis_index("s") * 2 + lax.axis_index("c")
        lane = lax.iota(jnp.int32, 16)
        nslot = 6
        n_ch = 18

        def radix_pass(ko, n, p):
            if p == 0:
                kin, iin = None, None
            elif p % 2 == 1:
                kin, iin = k0, i0
            else:
                kin, iin = k1, i1
            kout, iout = (k0, i0) if p % 2 == 0 else (k1, i1)
            sh = 8 * p

            def hz(i, _):
                off[pl.ds(i * 16, 16)] = jnp.zeros((16,), jnp.int32)
                return 0

            lax.fori_loop(0, 16, hz, 0)

            def hb(i, _):
                kk = keys_v[pl.ds(ko + i * 16, 16)] if p == 0 else (
                    kin[pl.ds(i * 16, 16)])
                d = lax.shift_right_logical(kk, sh) & 255
                occ, lastm = plsc.scan_count(d)
                plsc.addupdate_scatter(off, [d], occ, mask=lastm)
                return 0

            lax.fori_loop(0, n // 16, hb, 0)

            def pb(i, carry):
                h = off[pl.ds(i * 16, 16)]
                c = plsc.cumsum(h)
                off[pl.ds(i * 16, 16)] = c - h + carry
                return carry + jnp.sum(h)

            lax.fori_loop(0, 16, pb, jnp.int32(0))

            def mb(i, _):
                if p == 0:
                    kk = keys_v[pl.ds(ko + i * 16, 16)]
                    v = lane + i * 16
                else:
                    kk = kin[pl.ds(i * 16, 16)]
                    v = iin[pl.ds(i * 16, 16)]
                d = lax.shift_right_logical(kk, sh) & 255
                occ, lastm = plsc.scan_count(d)
                base = plsc.load_gather(off, [d])
                pos = base + occ - 1
                if p != 3:
                    plsc.store_scatter(kout, [pos], kk)
                plsc.store_scatter(iout, [pos], v)
                plsc.addupdate_scatter(off, [d], occ, mask=lastm)
                return 0

            lax.fori_loop(0, n // 16, mb, 0)

        def build_idx(dst, count, base):
            def bd(i, _):
                dst[pl.ds(i * 16, 16)] = i1[pl.ds(i * 16, 16)] + base
                return 0

            lax.fori_loop(0, count // 16, bd, 0)

        def build_idx_agent(dst, b):
            ab_base = b * (_H * _N_AG)
            for h in range(_H):
                def ai(i, _):
                    dst[pl.ds(h * _K_AG + i * 16, 16)] = (
                        i1[pl.ds(i * 16, 16)] + (ab_base + h * _N_AG))
                    return 0

                lax.fori_loop(0, _K_AG // 16, ai, 0)

        def compute_steps(bi):
            b = wid * nb + bi
            im, ip, ia = idx_m[bi % 2], idx_p[bi % 2], idx_a[bi % 2]
            return [
                lambda: (pltpu.sync_copy(
                    keys_h.at[pl.ds(b * _NK, _NK)], keys_v),
                    radix_pass(_KO_MAP, _N_MAP, 0))[-1],
                lambda: radix_pass(_KO_MAP, _N_MAP, 1),
                lambda: radix_pass(_KO_MAP, _N_MAP, 2),
                lambda: (radix_pass(_KO_MAP, _N_MAP, 3),
                         build_idx(im, _K_MAP, b * _N_MAP))[-1],
                lambda: (radix_pass(_KO_POLY, _N_POLY, 0),
                         radix_pass(_KO_POLY, _N_POLY, 1))[-1],
                lambda: (radix_pass(_KO_POLY, _N_POLY, 2),
                         radix_pass(_KO_POLY, _N_POLY, 3),
                         build_idx(ip, _K_POLY, b * _N_POLY))[-1],
                lambda: (radix_pass(_KO_AG, _N_AG, 0),
                         radix_pass(_KO_AG, _N_AG, 1))[-1],
                lambda: (radix_pass(_KO_AG, _N_AG, 2),
                         radix_pass(_KO_AG, _N_AG, 3),
                         build_idx_agent(ia, b))[-1],
            ]

        def chunks_for(bi):
            b = wid * nb + bi
            ob = b * _OUT
            im, ip, ia = idx_m[bi % 2], idx_p[bi % 2], idx_a[bi % 2]
            return (
                [("g", im, j, mf_h, ob + _OFF_MAP + j * 128, b)
                 for j in range(8)]
                + [("g", ip, j, pf_h, ob + _OFF_POLY + j * 128, b)
                   for j in range(4)]
                + [("g", ia, j, af_h, ob + j * 128, b) for j in range(4)]
                + [("r", None, j, rf_h, ob + _OFF_RT + j * 128, b)
                   for j in range(2)]
            )

        all_chunks = []
        for bi in range(nb):
            all_chunks.extend(chunks_for(bi))
        n_total = len(all_chunks)
        gh = {}
        wh = {}

        def fire(J):
            kind, idx_ref, jj, table, _, b = all_chunks[J]
            slot = buf.at[J % nslot]
            if kind == "g":
                gh[J] = pltpu.async_copy(
                    table.at[idx_ref.at[pl.ds(jj * 128, 128)]], slot, sem_g)
            else:
                gh[J] = pltpu.async_copy(
                    table.at[pl.ds(b * _N_RT + jj * 128, 128)], slot, sem_g)

        for step in compute_steps(0):
            step()
        for J in range(4):
            fire(J)

        for J in range(n_total):
            bi, j = divmod(J, n_ch)
            nxt = J + 4
            if nxt < n_total:
                if nxt >= nslot:
                    wh[nxt - nslot].wait()
                fire(nxt)
            gh[J].wait()
            wh[J] = pltpu.async_copy(
                buf.at[J % nslot],
                out_h.at[pl.ds(all_chunks[J][4], 128)], sem_o)
            if bi + 1 < nb and j % 2 == 0 and j // 2 < 8:
                compute_steps(bi + 1)[j // 2]()
        for J in range(n_total - nslot, n_total):
            wh[J].wait()

    return k(keys, af, mf, rf, pf)


def kernel(agent_feats, agent_poses, map_feats, map_poses, route_feats,
           polygon_feats, polygon_poses):
    B = agent_feats.shape[0]
    keys = _keys(agent_poses, map_poses, polygon_poses)
    out = _sc_gather(
        keys,
        agent_feats.reshape(B * _H * _N_AG, _D),
        map_feats.reshape(B * _N_MAP, _D),
        route_feats.reshape(B * _N_RT, _D),
        polygon_feats.reshape(B * _N_POLY, _D),
        B)
    return out.reshape(B, _OUT, _D)
